# baseline (speedup 1.0000x reference)
import functools

import jax
import jax.numpy as jnp
from jax import lax
from jax.experimental import pallas as pl
from jax.experimental.pallas import tpu as pltpu
from jax.experimental.pallas import tpu_sc as plsc

B = 16384
D = 128
RD = 128
NREL = 64
LAM = 1e-05

NC = 2
NS = 16
NW = NC * NS
ROWS_PER_W = B // NW
CHUNK = 128
NCHUNK = ROWS_PER_W // CHUNK

TILE = 256
NTILES = B // TILE
GROUP = 8
NGROUP = NREL // GROUP


def _sc_gather_body(table_hbm, h_hbm, p_hbm, n_hbm,
                    out_h, out_p, out_n,
                    idx_v, rows_v, sem):
    wid = lax.axis_index("s") * NC + lax.axis_index("c")
    base = wid * ROWS_PER_W
    for idx_hbm, out_hbm in ((h_hbm, out_h), (p_hbm, out_p), (n_hbm, out_n)):
        for c in range(NCHUNK):
            off = base + c * CHUNK
            pltpu.sync_copy(idx_hbm.at[pl.ds(off, CHUNK)], idx_v)
            pltpu.async_copy(table_hbm.at[idx_v], rows_v, sem).wait()
            pltpu.sync_copy(rows_v, out_hbm.at[pl.ds(off, CHUNK)])


def _sc_gather(entity_embed, h, p, n):
    mesh = plsc.VectorSubcoreMesh(core_axis_name="c", subcore_axis_name="s")
    f = pl.kernel(
        _sc_gather_body,
        out_type=(
            jax.ShapeDtypeStruct((B, D), jnp.float32),
            jax.ShapeDtypeStruct((B, D), jnp.float32),
            jax.ShapeDtypeStruct((B, D), jnp.float32),
        ),
        mesh=mesh,
        scratch_types=[
            pltpu.VMEM((CHUNK,), jnp.int32),
            pltpu.VMEM((CHUNK, D), jnp.float32),
            pltpu.SemaphoreType.DMA,
        ],
    )
    return f(entity_embed, h, p, n)


def _normalize(x):
    n = jnp.sqrt(jnp.sum(x * x, axis=1, keepdims=True))
    return x / jnp.maximum(n, 1e-12)


def _tc_body(r_ref, he_ref, pe_ref, ne_ref, rel_ref, wm_ref, out_ref):
    t = pl.program_id(0)
    r = r_ref[...]
    X = jnp.concatenate([he_ref[...], pe_ref[...], ne_ref[...]],
                        axis=0).astype(jnp.bfloat16)
    rr = jnp.concatenate([r, r, r], axis=0)
    X4 = jnp.concatenate([X] * GROUP, axis=1)
    lane_rel = lax.broadcasted_iota(
        jnp.int32, (3 * TILE, GROUP * D), 1) // D

    acc = jnp.zeros((3 * TILE, RD), jnp.float32)
    for j in range(NGROUP):
        mask = rr == (lane_rel + j * GROUP)
        Xm = jnp.where(mask, X4, jnp.bfloat16(0.0))
        acc = acc + jnp.dot(Xm, wm_ref[j],
                            preferred_element_type=jnp.float32)

    onehot = (r == lax.broadcasted_iota(jnp.int32, (TILE, NREL), 1))
    r_emb = jnp.dot(onehot.astype(jnp.float32), rel_ref[...],
                    preferred_element_type=jnp.float32)

    mh = _normalize(acc[:TILE])
    mp = _normalize(acc[TILE:2 * TILE])
    mn = _normalize(acc[2 * TILE:])
    re = _normalize(r_emb)

    base = mh + re
    pos = jnp.sqrt(jnp.sum((base - mp) ** 2, axis=1))
    neg = jnp.sqrt(jnp.sum((base - mn) ** 2, axis=1))
    kg = -jnp.log(1.0 / (1.0 + jnp.exp(pos - neg)) + 1e-08)
    l2 = 0.5 * (jnp.sum(mh * mh) + jnp.sum(re * re)
                + jnp.sum(mp * mp) + jnp.sum(mn * mn))
    partial = jnp.sum(kg) / B + LAM * l2 / B

    @pl.when(t == 0)
    def _init():
        out_ref[0, 0] = 0.0

    out_ref[0, 0] += partial


def _tc_loss(r2d, he, pe, ne, relation_embed, trans_M):
    return pl.pallas_call(
        _tc_body,
        grid=(NTILES,),
        in_specs=[
            pl.BlockSpec((TILE, 1), lambda t: (t, 0)),
            pl.BlockSpec((TILE, D), lambda t: (t, 0)),
            pl.BlockSpec((TILE, D), lambda t: (t, 0)),
            pl.BlockSpec((TILE, D), lambda t: (t, 0)),
            pl.BlockSpec((NREL, RD), lambda t: (0, 0)),
            pl.BlockSpec((NGROUP, GROUP * D, RD), lambda t: (0, 0, 0)),
        ],
        out_specs=pl.BlockSpec(memory_space=pltpu.SMEM),
        out_shape=jax.ShapeDtypeStruct((1, 1), jnp.float32),
        compiler_params=pltpu.CompilerParams(
            dimension_semantics=("arbitrary",),
        ),
    )(r2d, he, pe, ne, relation_embed, trans_M)


def kernel(h, r, pos_t, neg_t, entity_embed, relation_embed, trans_M):
    h = h.astype(jnp.int32)
    r = r.astype(jnp.int32)
    pos_t = pos_t.astype(jnp.int32)
    neg_t = neg_t.astype(jnp.int32)
    he, pe, ne = _sc_gather(entity_embed, h, pos_t, neg_t)
    wm4 = trans_M.astype(jnp.bfloat16).reshape(NGROUP, GROUP * D, RD)
    out = _tc_loss(r.reshape(B, 1), he, pe, ne, relation_embed, wm4)
    return out.reshape(())


# speedup vs baseline: 1.0310x; 1.0310x over previous
import functools

import jax
import jax.numpy as jnp
from jax import lax
from jax.experimental import pallas as pl
from jax.experimental.pallas import tpu as pltpu
from jax.experimental.pallas import tpu_sc as plsc

B = 16384
D = 128
RD = 128
NREL = 64
LAM = 1e-05

NC = 2
NS = 16
NW = NC * NS
ROWS_PER_W = B // NW
CHUNK = 128
NCHUNK = ROWS_PER_W // CHUNK

TILE = 512
NTILES = B // TILE
GROUP = 4
NGROUP = NREL // GROUP


def _sc_gather_body(table_hbm, h_hbm, p_hbm, n_hbm,
                    out_h, out_p, out_n,
                    idx_v, rows_v, sem):
    wid = lax.axis_index("s") * NC + lax.axis_index("c")
    base = wid * ROWS_PER_W
    for idx_hbm, out_hbm in ((h_hbm, out_h), (p_hbm, out_p), (n_hbm, out_n)):
        for c in range(NCHUNK):
            off = base + c * CHUNK
            pltpu.sync_copy(idx_hbm.at[pl.ds(off, CHUNK)], idx_v)
            pltpu.async_copy(table_hbm.at[idx_v], rows_v, sem).wait()
            pltpu.sync_copy(rows_v, out_hbm.at[pl.ds(off, CHUNK)])


def _sc_gather(entity_embed, h, p, n):
    mesh = plsc.VectorSubcoreMesh(core_axis_name="c", subcore_axis_name="s")
    f = pl.kernel(
        _sc_gather_body,
        out_type=(
            jax.ShapeDtypeStruct((B, D), jnp.float32),
            jax.ShapeDtypeStruct((B, D), jnp.float32),
            jax.ShapeDtypeStruct((B, D), jnp.float32),
        ),
        mesh=mesh,
        scratch_types=[
            pltpu.VMEM((CHUNK,), jnp.int32),
            pltpu.VMEM((CHUNK, D), jnp.float32),
            pltpu.SemaphoreType.DMA,
        ],
    )
    return f(entity_embed, h, p, n)


def _normalize(x):
    n = jnp.sqrt(jnp.sum(x * x, axis=1, keepdims=True))
    return x / jnp.maximum(n, 1e-12)


def _tc_body(r_ref, he_ref, pe_ref, ne_ref, rel_ref, wm_ref, out_ref):
    t = pl.program_id(0)
    r = r_ref[...]
    X = jnp.concatenate([he_ref[...], pe_ref[...], ne_ref[...]],
                        axis=0).astype(jnp.bfloat16)
    rr = jnp.concatenate([r, r, r], axis=0)
    X4 = jnp.concatenate([X] * GROUP, axis=1)
    lane_rel = lax.broadcasted_iota(
        jnp.int32, (3 * TILE, GROUP * D), 1) // D

    acc = jnp.zeros((3 * TILE, RD), jnp.float32)
    for j in range(NGROUP):
        mask = rr == (lane_rel + j * GROUP)
        Xm = jnp.where(mask, X4, jnp.bfloat16(0.0))
        acc = acc + jnp.dot(Xm, wm_ref[j],
                            preferred_element_type=jnp.float32)

    onehot = (r == lax.broadcasted_iota(jnp.int32, (TILE, NREL), 1))
    r_emb = jnp.dot(onehot.astype(jnp.float32), rel_ref[...],
                    preferred_element_type=jnp.float32)

    mh = _normalize(acc[:TILE])
    mp = _normalize(acc[TILE:2 * TILE])
    mn = _normalize(acc[2 * TILE:])
    re = _normalize(r_emb)

    base = mh + re
    pos = jnp.sqrt(jnp.sum((base - mp) ** 2, axis=1))
    neg = jnp.sqrt(jnp.sum((base - mn) ** 2, axis=1))
    kg = -jnp.log(1.0 / (1.0 + jnp.exp(pos - neg)) + 1e-08)
    l2 = 0.5 * (jnp.sum(mh * mh) + jnp.sum(re * re)
                + jnp.sum(mp * mp) + jnp.sum(mn * mn))
    partial = jnp.sum(kg) / B + LAM * l2 / B

    @pl.when(t == 0)
    def _init():
        out_ref[0, 0] = 0.0

    out_ref[0, 0] += partial


def _tc_loss(r2d, he, pe, ne, relation_embed, trans_M):
    return pl.pallas_call(
        _tc_body,
        grid=(NTILES,),
        in_specs=[
            pl.BlockSpec((TILE, 1), lambda t: (t, 0)),
            pl.BlockSpec((TILE, D), lambda t: (t, 0)),
            pl.BlockSpec((TILE, D), lambda t: (t, 0)),
            pl.BlockSpec((TILE, D), lambda t: (t, 0)),
            pl.BlockSpec((NREL, RD), lambda t: (0, 0)),
            pl.BlockSpec((NGROUP, GROUP * D, RD), lambda t: (0, 0, 0)),
        ],
        out_specs=pl.BlockSpec(memory_space=pltpu.SMEM),
        out_shape=jax.ShapeDtypeStruct((1, 1), jnp.float32),
        compiler_params=pltpu.CompilerParams(
            dimension_semantics=("arbitrary",),
        ),
    )(r2d, he, pe, ne, relation_embed, trans_M)


def kernel(h, r, pos_t, neg_t, entity_embed, relation_embed, trans_M):
    h = h.astype(jnp.int32)
    r = r.astype(jnp.int32)
    pos_t = pos_t.astype(jnp.int32)
    neg_t = neg_t.astype(jnp.int32)
    he, pe, ne = _sc_gather(entity_embed, h, pos_t, neg_t)
    wm4 = trans_M.astype(jnp.bfloat16).reshape(NGROUP, GROUP * D, RD)
    out = _tc_loss(r.reshape(B, 1), he, pe, ne, relation_embed, wm4)
    return out.reshape(())
